# baseline (speedup 1.0000x reference)
import functools

import jax
import jax.numpy as jnp
from jax import lax
from jax.experimental import pallas as pl
from jax.experimental.pallas import tpu as pltpu
from jax.experimental.pallas import tpu_sc as plsc

BATCH = 4096
SEQ = 200
EMBED_DIM = 64
TOTAL = BATCH * SEQ
NUM_CORES = 2
NUM_SUBCORES = 16
NW = NUM_CORES * NUM_SUBCORES
B_PER_W = TOTAL // NW
IDXV = 128
KSUB = 4
CHUNK = IDXV * KSUB
NSTEPS = B_PER_W // CHUNK

_mesh = plsc.VectorSubcoreMesh(core_axis_name="c", subcore_axis_name="s")


@functools.partial(
    pl.kernel,
    out_type=jax.ShapeDtypeStruct((TOTAL, EMBED_DIM), jnp.float32),
    mesh=_mesh,
    compiler_params=pltpu.CompilerParams(use_tc_tiling_on_sc=False),
    scratch_types=[
        pltpu.VMEM((NSTEPS * KSUB, IDXV), jnp.int32),
        pltpu.VMEM((CHUNK, EMBED_DIM), jnp.float32),
        pltpu.VMEM((CHUNK, EMBED_DIM), jnp.float32),
        pltpu.SemaphoreType.DMA,
        pltpu.SemaphoreType.DMA,
    ],
)
def _sc_gather(idx_hbm, table_hbm, out_hbm, idx_v, rows0, rows1, sem0, sem1):
    wid = lax.axis_index("s") * NUM_CORES + lax.axis_index("c")
    base = wid * B_PER_W

    pltpu.sync_copy(idx_hbm.at[wid], idx_v)

    rows = (rows0, rows1)
    sems = (sem0, sem1)

    def _start(step, b):
        for j in range(KSUB):
            pltpu.async_copy(
                table_hbm.at[idx_v.at[step * KSUB + j]],
                rows[b].at[pl.ds(j * IDXV, IDXV)],
                sems[b],
            )

    def _finish(step, b):
        for j in range(KSUB):
            pltpu.make_async_copy(
                table_hbm.at[idx_v.at[step * KSUB + j]],
                rows[b].at[pl.ds(j * IDXV, IDXV)],
                sems[b],
            ).wait()
        pltpu.sync_copy(rows[b], out_hbm.at[pl.ds(base + step * CHUNK, CHUNK)])

    _start(0, 0)

    def body(i, _):
        g = 2 * i
        _start(g + 1, 1)
        _finish(g, 0)

        @pl.when(g + 2 < NSTEPS)
        def _():
            _start(g + 2, 0)

        _finish(g + 1, 1)
        return _

    lax.fori_loop(0, NSTEPS // 2, body, None)


def kernel(input_ids, word_embeddings):
    idx = input_ids.astype(jnp.int32).reshape(NW, NSTEPS * KSUB, IDXV)
    out = _sc_gather(idx, word_embeddings)
    return out.reshape(BATCH, SEQ, EMBED_DIM)
